# baseline (speedup 1.0000x reference)
import functools

import jax
import jax.numpy as jnp
from jax import lax
from jax.experimental import pallas as pl
from jax.experimental.pallas import tpu as pltpu
from jax.experimental.pallas import tpu_sc as plsc

_NUM_WORKERS = 32
_CHUNK = 1024
_PACK = 8
_BB = 512
_SLICES = 5


def _tc_repack(x, k, blk_rows):
    rows_total, m = x.shape
    grid = rows_total // blk_rows
    out_rows = blk_rows // k
    assert grid * blk_rows == rows_total and out_rows * k == blk_rows

    def body(x_ref, o_ref):
        xb = x_ref[...].reshape(blk_rows // k, k, m)
        o_ref[...] = jnp.concatenate([xb[:, j, :] for j in range(k)], axis=1)

    return pl.pallas_call(
        body,
        grid=(grid,),
        in_specs=[pl.BlockSpec((blk_rows, m), lambda i: (i, 0))],
        out_specs=pl.BlockSpec((out_rows, m * k), lambda i: (i, 0)),
        out_shape=jax.ShapeDtypeStruct((rows_total // k, m * k), x.dtype),
    )(x)


def _sc_gather(table, ids_flat):
    n = ids_flat.shape[0]
    e = table.shape[1]
    per_w = n // _NUM_WORKERS
    n_chunks = per_w // _CHUNK
    assert per_w * _NUM_WORKERS == n and n_chunks * _CHUNK == per_w

    mesh = plsc.VectorSubcoreMesh(core_axis_name="c", subcore_axis_name="s")

    @functools.partial(
        pl.kernel,
        mesh=mesh,
        out_type=jax.ShapeDtypeStruct((n, e), jnp.float32),
        scratch_types=[
            pltpu.VMEM((_CHUNK,), jnp.int32),
            pltpu.VMEM((_CHUNK, e), jnp.float32),
            pltpu.SemaphoreType.DMA,
        ],
        compiler_params=pltpu.CompilerParams(use_tc_tiling_on_sc=False),
    )
    def gather_kernel(table_hbm, ids_hbm, out_hbm, idx_v, rows_v, sem):
        wid = lax.axis_index("s") * 2 + lax.axis_index("c")
        base = wid * per_w

        def body(i, carry):
            off = base + i * _CHUNK
            pltpu.sync_copy(ids_hbm.at[pl.ds(off, _CHUNK)], idx_v)
            pltpu.async_copy(table_hbm.at[idx_v], rows_v, sem).wait()
            pltpu.sync_copy(rows_v, out_hbm.at[pl.ds(off, _CHUNK)])
            return carry

        lax.fori_loop(0, n_chunks, body, 0)

    return gather_kernel(table, ids_flat)


def _tc_project_slice(w_packed, x_packed, prev, lg0, lgs, l_groups, b):
    kdim = w_packed.shape[1]
    out_rows = w_packed.shape[0]
    h = out_rows // _PACK
    b_blocks = b // _BB

    def body(w_ref, x_ref, *refs):
        o_ref = refs[-1]
        prod = jax.lax.dot_general(
            w_ref[...],
            x_ref[...],
            dimension_numbers=(((1,), (1,)), ((), ())),
            preferred_element_type=jnp.float32,
        )
        o_ref[...] = prod.reshape(_PACK, h, _BB)

    in_specs = [
        pl.BlockSpec((out_rows, kdim), lambda lg, bb: (0, 0)),
        pl.BlockSpec((_BB, kdim), lambda lg, bb: (lg * b_blocks + bb, 0)),
    ]
    args = [w_packed, x_packed]
    aliases = {}
    if prev is not None:
        in_specs.append(
            pl.BlockSpec((_PACK, h, _BB), lambda lg, bb: (0, 0, 0))
        )
        args.append(prev)
        aliases = {2: 0}

    return pl.pallas_call(
        body,
        grid=(lgs, b_blocks),
        in_specs=in_specs,
        out_specs=pl.BlockSpec(
            (_PACK, h, _BB), lambda lg, bb: (lg0 + lg, 0, bb)
        ),
        out_shape=jax.ShapeDtypeStruct((l_groups * _PACK, h, b), jnp.float32),
        input_output_aliases=aliases,
        compiler_params=pltpu.CompilerParams(
            dimension_semantics=("arbitrary", "arbitrary"),
        ),
    )(*args)


def kernel(input_ids, embeddings_VE, linear_EH_weight):
    b, l = input_ids.shape
    e = embeddings_VE.shape[1]
    h = linear_EH_weight.shape[0]
    n = b * l
    l_groups = l // _PACK
    assert l_groups % _SLICES == 0
    lgs = l_groups // _SLICES
    n_s = n // _SLICES

    ids3t = input_ids.T.reshape(l_groups, _PACK, b).transpose(0, 2, 1)
    ids_perm = _tc_repack(
        ids3t.reshape(l_groups * b, _PACK), 16, b
    ).reshape(n)
    table_packed = _tc_repack(embeddings_VE, _PACK, 8000)
    table_rm = table_packed.reshape(embeddings_VE.shape)
    w_packed = jnp.kron(
        jnp.eye(_PACK, dtype=jnp.float32), linear_EH_weight
    )

    out = None
    for s in range(_SLICES):
        ids_s = lax.slice(ids_perm, (s * n_s,), ((s + 1) * n_s,))
        rows_s = _sc_gather(table_rm, ids_s)
        x_s = rows_s.reshape(n_s // _PACK, _PACK * e)
        out = _tc_project_slice(
            w_packed, x_s, out, s * lgs, lgs, l_groups, b
        )

    return out.transpose(2, 0, 1)


# speedup vs baseline: 1.3197x; 1.3197x over previous
import functools

import jax
import jax.numpy as jnp
from jax import lax
from jax.experimental import pallas as pl
from jax.experimental.pallas import tpu as pltpu
from jax.experimental.pallas import tpu_sc as plsc

_NUM_WORKERS = 32
_CHUNK = 1024
_PACK = 8
_BB = 2048
_SLICES = 5


def _sc_gather(table, ids_flat):
    n = ids_flat.shape[0]
    e = table.shape[1]
    per_w = n // _NUM_WORKERS
    n_chunks = per_w // _CHUNK
    assert per_w * _NUM_WORKERS == n and n_chunks * _CHUNK == per_w

    mesh = plsc.VectorSubcoreMesh(core_axis_name="c", subcore_axis_name="s")

    @functools.partial(
        pl.kernel,
        mesh=mesh,
        out_type=jax.ShapeDtypeStruct((n, e), jnp.float32),
        scratch_types=[
            pltpu.VMEM((_CHUNK,), jnp.int32),
            pltpu.VMEM((_CHUNK, e), jnp.float32),
            pltpu.SemaphoreType.DMA,
        ],
        compiler_params=pltpu.CompilerParams(use_tc_tiling_on_sc=False),
    )
    def gather_kernel(table_hbm, ids_hbm, out_hbm, idx_v, rows_v, sem):
        wid = lax.axis_index("s") * 2 + lax.axis_index("c")
        base = wid * per_w

        def body(i, carry):
            off = base + i * _CHUNK
            pltpu.sync_copy(ids_hbm.at[pl.ds(off, _CHUNK)], idx_v)
            pltpu.async_copy(table_hbm.at[idx_v], rows_v, sem).wait()
            pltpu.sync_copy(rows_v, out_hbm.at[pl.ds(off, _CHUNK)])
            return carry

        lax.fori_loop(0, n_chunks, body, 0)

    return gather_kernel(table, ids_flat)


def _tc_project_slice(w_packed, x_packed, prev, lg0, lgs, l_groups, b):
    kdim = w_packed.shape[1]
    out_rows = w_packed.shape[0]
    h = out_rows // _PACK
    b_blocks = b // _BB

    def body(w_ref, x_ref, *refs):
        o_ref = refs[-1]
        prod = jax.lax.dot_general(
            w_ref[...],
            x_ref[...],
            dimension_numbers=(((1,), (1,)), ((), ())),
            preferred_element_type=jnp.float32,
        )
        o_ref[...] = prod.reshape(_PACK, h, _BB)

    in_specs = [
        pl.BlockSpec((out_rows, kdim), lambda lg, bb: (0, 0)),
        pl.BlockSpec((_BB, kdim), lambda lg, bb: (lg * b_blocks + bb, 0)),
    ]
    args = [w_packed, x_packed]
    aliases = {}
    if prev is not None:
        in_specs.append(
            pl.BlockSpec((_PACK, h, _BB), lambda lg, bb: (0, 0, 0))
        )
        args.append(prev)
        aliases = {2: 0}

    return pl.pallas_call(
        body,
        grid=(lgs, b_blocks),
        in_specs=in_specs,
        out_specs=pl.BlockSpec(
            (_PACK, h, _BB), lambda lg, bb: (lg0 + lg, 0, bb)
        ),
        out_shape=jax.ShapeDtypeStruct((l_groups * _PACK, h, b), jnp.float32),
        input_output_aliases=aliases,
        compiler_params=pltpu.CompilerParams(
            dimension_semantics=("arbitrary", "arbitrary"),
        ),
    )(*args)


def kernel(input_ids, embeddings_VE, linear_EH_weight):
    b, l = input_ids.shape
    e = embeddings_VE.shape[1]
    h = linear_EH_weight.shape[0]
    n = b * l
    l_groups = l // _PACK
    assert l_groups % _SLICES == 0
    lgs = l_groups // _SLICES
    n_s = n // _SLICES

    ids_perm = (
        input_ids.T.reshape(l_groups, _PACK, b)
        .transpose(0, 2, 1)
        .reshape(n)
    )
    w_packed = jnp.kron(
        jnp.eye(_PACK, dtype=jnp.float32), linear_EH_weight
    )

    out = None
    for s in range(_SLICES):
        ids_s = lax.slice(ids_perm, (s * n_s,), ((s + 1) * n_s,))
        rows_s = _sc_gather(embeddings_VE, ids_s)
        x_s = rows_s.reshape(n_s // _PACK, _PACK * e)
        out = _tc_project_slice(
            w_packed, x_s, out, s * lgs, lgs, l_groups, b
        )

    return out.transpose(2, 0, 1)
